# Initial kernel scaffold; baseline (speedup 1.0000x reference)
import jax
import jax.numpy as jnp
from jax.experimental import pallas as pl


def kernel(mem_input, mem, ts, mem_ts, h, time_w, time_b, W_ih, W_hh, b_ih, b_hh, W_map, b_map):
    raise NotImplementedError("write your pallas kernel here")


# speedup vs baseline: 1.2986x; 1.2986x over previous
import functools

import jax
import jax.numpy as jnp
from jax.experimental import pallas as pl

N = 10000
DIN = 256
DH = 256
DT = 100
DTP = 128
DNF = 512
BLK = 2000


@functools.partial(jax.jit, static_argnames=("interpret",))
def _run(mem_input, mem, ts, mem_ts, h, time_w, time_b,
         W_ih, W_hh, b_ih, b_hh, W_map, b_map, interpret=False):
    w1 = W_ih[:, :DIN].T
    w2 = jnp.zeros((DTP, 3 * DH), jnp.float32).at[:DT].set(W_ih[:, DIN:].T)
    wh = W_hh.T
    wm = W_map.T
    tw = jnp.zeros((1, DTP), jnp.float32).at[0, :DT].set(time_w + 0.0)
    tb = jnp.zeros((1, DTP), jnp.float32).at[0, :DT].set(time_b)

    grid = (N // BLK,)
    row = lambda i: (i, 0)
    rep = lambda i: (0, 0)

    def kern(ts_ref, mts_ref, tb_ref, x_ref, mem_ref, h_ref,
             tw_ref, w1_ref, w2_ref, wh_ref, wm_ref,
             bih_ref, bhh_ref, bmap_ref, out_ref):
        dt = ts_ref[...] - mts_ref[...]
        tf = jnp.cos(dt * tw_ref[...] + tb_ref[...])
        x = x_ref[...]
        mem = mem_ref[...]
        gi = (jnp.dot(x, w1_ref[...], preferred_element_type=jnp.float32)
              + jnp.dot(tf, w2_ref[...], preferred_element_type=jnp.float32)
              + bih_ref[...])
        gh = (jnp.dot(mem, wh_ref[...], preferred_element_type=jnp.float32)
              + bhh_ref[...])
        r = jax.nn.sigmoid(gi[:, 0:DH] + gh[:, 0:DH])
        z = jax.nn.sigmoid(gi[:, DH:2 * DH] + gh[:, DH:2 * DH])
        n = jnp.tanh(gi[:, 2 * DH:3 * DH] + r * gh[:, 2 * DH:3 * DH])
        memory = (1.0 - z) * n + z * mem
        out_ref[...] = (memory
                        + jnp.dot(h_ref[...], wm_ref[...],
                                  preferred_element_type=jnp.float32)
                        + bmap_ref[...])

    return pl.pallas_call(
        kern,
        grid=grid,
        in_specs=[
            pl.BlockSpec((BLK, 1), row),
            pl.BlockSpec((BLK, 1), row),
            pl.BlockSpec((1, DTP), rep),
            pl.BlockSpec((BLK, DIN), row),
            pl.BlockSpec((BLK, DH), row),
            pl.BlockSpec((BLK, DNF), row),
            pl.BlockSpec((1, DTP), rep),
            pl.BlockSpec((DIN, 3 * DH), rep),
            pl.BlockSpec((DTP, 3 * DH), rep),
            pl.BlockSpec((DH, 3 * DH), rep),
            pl.BlockSpec((DNF, DH), rep),
            pl.BlockSpec((1, 3 * DH), rep),
            pl.BlockSpec((1, 3 * DH), rep),
            pl.BlockSpec((1, DH), rep),
        ],
        out_specs=pl.BlockSpec((BLK, DH), row),
        out_shape=jax.ShapeDtypeStruct((N, DH), jnp.float32),
        interpret=interpret,
    )(ts.reshape(N, 1), mem_ts.reshape(N, 1), tb, mem_input, mem, h,
      tw, w1, w2, wh, wm,
      b_ih.reshape(1, -1), b_hh.reshape(1, -1), b_map.reshape(1, -1))


def kernel(mem_input, mem, ts, mem_ts, h, time_w, time_b,
           W_ih, W_hh, b_ih, b_hh, W_map, b_map):
    return _run(mem_input, mem, ts, mem_ts, h, time_w, time_b,
                W_ih, W_hh, b_ih, b_hh, W_map, b_map)


# speedup vs baseline: 1.3013x; 1.0021x over previous
import functools

import jax
import jax.numpy as jnp
from jax.experimental import pallas as pl

N = 10000
DIN = 256
DH = 256
DT = 100
DTP = 128
DNF = 512
BLK = 2000


@functools.partial(jax.jit, static_argnames=("interpret",))
def _run(mem_input, mem, ts, mem_ts, h, time_w, time_b,
         W_ih, W_hh, b_ih, b_hh, W_map, b_map, interpret=False):
    w1 = W_ih[:, :DIN].T
    w2 = jnp.zeros((DTP, 3 * DH), jnp.float32).at[:DT].set(W_ih[:, DIN:].T)
    wh = W_hh.T
    wm = W_map.T
    tw = jnp.zeros((1, DTP), jnp.float32).at[0, :DT].set(time_w + 0.0)
    tb = jnp.zeros((1, DTP), jnp.float32).at[0, :DT].set(time_b)

    grid = (N // BLK,)
    row = lambda i: (i, 0)
    rep = lambda i: (0, 0)

    def kern(ts_ref, mts_ref, tb_ref, x_ref, mem_ref, h_ref,
             tw_ref, w1_ref, w2_ref, wh_ref, wm_ref,
             bih_ref, bhh_ref, bmap_ref, out_ref):
        dt = ts_ref[...] - mts_ref[...]
        tf = jnp.cos(dt * tw_ref[...] + tb_ref[...])
        x = x_ref[...]
        mem = mem_ref[...]
        bf = jnp.bfloat16
        gi = (jnp.dot(x.astype(bf), w1_ref[...].astype(bf),
                      preferred_element_type=jnp.float32)
              + jnp.dot(tf.astype(bf), w2_ref[...].astype(bf),
                        preferred_element_type=jnp.float32)
              + bih_ref[...])
        gh = (jnp.dot(mem.astype(bf), wh_ref[...].astype(bf),
                      preferred_element_type=jnp.float32)
              + bhh_ref[...])
        r = jax.nn.sigmoid(gi[:, 0:DH] + gh[:, 0:DH])
        z = jax.nn.sigmoid(gi[:, DH:2 * DH] + gh[:, DH:2 * DH])
        n = jnp.tanh(gi[:, 2 * DH:3 * DH] + r * gh[:, 2 * DH:3 * DH])
        memory = (1.0 - z) * n + z * mem
        out_ref[...] = (memory
                        + jnp.dot(h_ref[...].astype(bf), wm_ref[...].astype(bf),
                                  preferred_element_type=jnp.float32)
                        + bmap_ref[...])

    return pl.pallas_call(
        kern,
        grid=grid,
        in_specs=[
            pl.BlockSpec((BLK, 1), row),
            pl.BlockSpec((BLK, 1), row),
            pl.BlockSpec((1, DTP), rep),
            pl.BlockSpec((BLK, DIN), row),
            pl.BlockSpec((BLK, DH), row),
            pl.BlockSpec((BLK, DNF), row),
            pl.BlockSpec((1, DTP), rep),
            pl.BlockSpec((DIN, 3 * DH), rep),
            pl.BlockSpec((DTP, 3 * DH), rep),
            pl.BlockSpec((DH, 3 * DH), rep),
            pl.BlockSpec((DNF, DH), rep),
            pl.BlockSpec((1, 3 * DH), rep),
            pl.BlockSpec((1, 3 * DH), rep),
            pl.BlockSpec((1, DH), rep),
        ],
        out_specs=pl.BlockSpec((BLK, DH), row),
        out_shape=jax.ShapeDtypeStruct((N, DH), jnp.float32),
        interpret=interpret,
    )(ts.reshape(N, 1), mem_ts.reshape(N, 1), tb, mem_input, mem, h,
      tw, w1, w2, wh, wm,
      b_ih.reshape(1, -1), b_hh.reshape(1, -1), b_map.reshape(1, -1))


def kernel(mem_input, mem, ts, mem_ts, h, time_w, time_b,
           W_ih, W_hh, b_ih, b_hh, W_map, b_map):
    return _run(mem_input, mem, ts, mem_ts, h, time_w, time_b,
                W_ih, W_hh, b_ih, b_hh, W_map, b_map)


# speedup vs baseline: 1.6143x; 1.2406x over previous
import functools

import jax
import jax.numpy as jnp
from jax.experimental import pallas as pl

N = 10000
DIN = 256
DH = 256
DT = 100
DTP = 128
DNF = 512
BLK = 2000


@functools.partial(jax.jit, static_argnames=("interpret",))
def _run(mem_input, mem, ts, mem_ts, h, time_w, time_b,
         W_ih, W_hh, b_ih, b_hh, W_map, b_map, interpret=False):
    w1 = W_ih[:, :DIN].T
    w2 = jnp.zeros((DTP, 3 * DH), jnp.float32).at[:DT].set(W_ih[:, DIN:].T)
    wh = W_hh.T
    wm = W_map.T
    tw = jnp.zeros((1, DTP), jnp.float32).at[0, :DT].set(time_w + 0.0)
    tb = jnp.zeros((1, DTP), jnp.float32).at[0, :DT].set(time_b)

    grid = (N // BLK,)
    row = lambda i: (i, 0)
    rep = lambda i: (0, 0)

    def kern(ts_ref, mts_ref, tb_ref, x_ref, mem_ref, h_ref,
             tw_ref, w1_ref, w2_ref, wh_ref, wm_ref,
             bih_ref, bhh_ref, bmap_ref, out_ref):
        dt = ts_ref[...] - mts_ref[...]
        arg = dt * tw_ref[...] + tb_ref[...]
        y = arg * 0.15915494309189535
        fr = y - jnp.round(y)
        s = fr * fr
        tf = (0.9999994436793983
              + s * (-19.739034372931126
                     + s * (64.93061336990448
                            + s * (-85.29597096153826
                                   + s * (58.912555324414804
                                          + s * -21.28302159300549)))))
        x = x_ref[...]
        mem = mem_ref[...]
        gi = (jnp.dot(x, w1_ref[...], preferred_element_type=jnp.float32)
              + jnp.dot(tf, w2_ref[...], preferred_element_type=jnp.float32)
              + bih_ref[...])
        gh = (jnp.dot(mem, wh_ref[...], preferred_element_type=jnp.float32)
              + bhh_ref[...])
        r = 0.5 + 0.5 * jnp.tanh(0.5 * (gi[:, 0:DH] + gh[:, 0:DH]))
        z = 0.5 + 0.5 * jnp.tanh(0.5 * (gi[:, DH:2 * DH] + gh[:, DH:2 * DH]))
        n = jnp.tanh(gi[:, 2 * DH:3 * DH] + r * gh[:, 2 * DH:3 * DH])
        memory = (1.0 - z) * n + z * mem
        out_ref[...] = (memory
                        + jnp.dot(h_ref[...], wm_ref[...],
                                  preferred_element_type=jnp.float32)
                        + bmap_ref[...])

    return pl.pallas_call(
        kern,
        grid=grid,
        in_specs=[
            pl.BlockSpec((BLK, 1), row),
            pl.BlockSpec((BLK, 1), row),
            pl.BlockSpec((1, DTP), rep),
            pl.BlockSpec((BLK, DIN), row),
            pl.BlockSpec((BLK, DH), row),
            pl.BlockSpec((BLK, DNF), row),
            pl.BlockSpec((1, DTP), rep),
            pl.BlockSpec((DIN, 3 * DH), rep),
            pl.BlockSpec((DTP, 3 * DH), rep),
            pl.BlockSpec((DH, 3 * DH), rep),
            pl.BlockSpec((DNF, DH), rep),
            pl.BlockSpec((1, 3 * DH), rep),
            pl.BlockSpec((1, 3 * DH), rep),
            pl.BlockSpec((1, DH), rep),
        ],
        out_specs=pl.BlockSpec((BLK, DH), row),
        out_shape=jax.ShapeDtypeStruct((N, DH), jnp.float32),
        interpret=interpret,
    )(ts.reshape(N, 1), mem_ts.reshape(N, 1), tb, mem_input, mem, h,
      tw, w1, w2, wh, wm,
      b_ih.reshape(1, -1), b_hh.reshape(1, -1), b_map.reshape(1, -1))


def kernel(mem_input, mem, ts, mem_ts, h, time_w, time_b,
           W_ih, W_hh, b_ih, b_hh, W_map, b_map):
    return _run(mem_input, mem, ts, mem_ts, h, time_w, time_b,
                W_ih, W_hh, b_ih, b_hh, W_map, b_map)


# speedup vs baseline: 1.8900x; 1.1707x over previous
import functools

import jax
import jax.numpy as jnp
from jax import lax
from jax.experimental import pallas as pl

N = 10000
DIN = 256
DH = 256
DT = 100
DNF = 512
BLK = 2000

_DN = (((1,), (1,)), ((), ()))


def _fused(ts_ref, mts_ref, x_ref, mem_ref, h_ref,
           tw_ref, tb_ref, wih_ref, whh_ref, wm_ref,
           bih_ref, bhh_ref, bmap_ref, out_ref):
    dt = ts_ref[...] - mts_ref[...]
    arg = dt * tw_ref[...] + tb_ref[...]
    y = arg * 0.15915494309189535
    fr = y - jnp.round(y)
    s = fr * fr
    tf = (0.9999994436793983
          + s * (-19.739034372931126
                 + s * (64.93061336990448
                        + s * (-85.29597096153826
                               + s * (58.912555324414804
                                      + s * -21.28302159300549)))))
    x = x_ref[...]
    mem = mem_ref[...]
    gi = (lax.dot_general(x, wih_ref[:, 0:DIN], _DN,
                          preferred_element_type=jnp.float32)
          + lax.dot_general(tf, wih_ref[:, DIN:DIN + DT], _DN,
                            preferred_element_type=jnp.float32)
          + bih_ref[...])
    gh = (lax.dot_general(mem, whh_ref[...], _DN,
                          preferred_element_type=jnp.float32)
          + bhh_ref[...])
    r = 0.5 + 0.5 * jnp.tanh(0.5 * (gi[:, 0:DH] + gh[:, 0:DH]))
    z = 0.5 + 0.5 * jnp.tanh(0.5 * (gi[:, DH:2 * DH] + gh[:, DH:2 * DH]))
    n = jnp.tanh(gi[:, 2 * DH:3 * DH] + r * gh[:, 2 * DH:3 * DH])
    memory = (1.0 - z) * n + z * mem
    out_ref[...] = (memory
                    + lax.dot_general(h_ref[...], wm_ref[...], _DN,
                                      preferred_element_type=jnp.float32)
                    + bmap_ref[...])


@functools.partial(jax.jit, static_argnames=("interpret",))
def _run(mem_input, mem, ts, mem_ts, h, time_w, time_b,
         W_ih, W_hh, b_ih, b_hh, W_map, b_map, interpret=False):
    grid = (N // BLK,)
    row = lambda i: (i, 0)
    rep = lambda i: (0, 0)
    return pl.pallas_call(
        _fused,
        grid=grid,
        in_specs=[
            pl.BlockSpec((BLK, 1), row),
            pl.BlockSpec((BLK, 1), row),
            pl.BlockSpec((BLK, DIN), row),
            pl.BlockSpec((BLK, DH), row),
            pl.BlockSpec((BLK, DNF), row),
            pl.BlockSpec((1, DT), rep),
            pl.BlockSpec((1, DT), rep),
            pl.BlockSpec((3 * DH, DIN + DT), rep),
            pl.BlockSpec((3 * DH, DH), rep),
            pl.BlockSpec((DH, DNF), rep),
            pl.BlockSpec((1, 3 * DH), rep),
            pl.BlockSpec((1, 3 * DH), rep),
            pl.BlockSpec((1, DH), rep),
        ],
        out_specs=pl.BlockSpec((BLK, DH), row),
        out_shape=jax.ShapeDtypeStruct((N, DH), jnp.float32),
        interpret=interpret,
    )(ts.reshape(N, 1), mem_ts.reshape(N, 1), mem_input, mem, h,
      time_w.reshape(1, DT), time_b.reshape(1, DT), W_ih, W_hh, W_map,
      b_ih.reshape(1, -1), b_hh.reshape(1, -1), b_map.reshape(1, -1))


def kernel(mem_input, mem, ts, mem_ts, h, time_w, time_b,
           W_ih, W_hh, b_ih, b_hh, W_map, b_map):
    return _run(mem_input, mem, ts, mem_ts, h, time_w, time_b,
                W_ih, W_hh, b_ih, b_hh, W_map, b_map)
